# baseline (speedup 1.0000x reference)
import functools

import jax
import jax.numpy as jnp
from jax import lax
from jax.experimental import pallas as pl
from jax.experimental.pallas import tpu as pltpu
from jax.experimental.pallas import tpu_sc as plsc

_B = 32
_H = 1024
_HH = 512
_N = 100000
_K = 64
_BN = 8192
_NBLK = 13
_NB1 = 7
_NB2 = _NBLK - _NB1
_W1 = _NB1 * _BN
_W2C = _NB2 * _BN
_CH = 256


def _mk_scores_body(off_blk, has_alias):
    def body(*refs):
        if has_alias:
            h_ref, w2_ref, b2_ref, comp_ref, ema_ref, _, out_ref, exact_ref = refs
        else:
            h_ref, w2_ref, b2_ref, comp_ref, ema_ref, out_ref, exact_ref = refs
        s = jnp.dot(h_ref[...], w2_ref[...],
                    preferred_element_type=jnp.float32) + b2_ref[...]
        s = s + comp_ref[...] * 0.3 + (1.0 / (ema_ref[...] + 1e-6)) * 0.1
        col = ((off_blk + pl.program_id(0)) * _BN
               + lax.broadcasted_iota(jnp.int32, (_B, _BN), 1))
        out_ref[...] = jnp.where(col < _N, s, -jnp.inf)
        exact_ref[...] = s
    return body


def _tc_part1(h, W2, b2, comp, ema):
    return pl.pallas_call(
        _mk_scores_body(0, False),
        grid=(_NB1,),
        in_specs=[
            pl.BlockSpec((_B, _HH), lambda j: (0, 0)),
            pl.BlockSpec((_HH, _BN), lambda j: (0, j)),
            pl.BlockSpec((1, _BN), lambda j: (0, j)),
            pl.BlockSpec((1, _BN), lambda j: (0, j)),
            pl.BlockSpec((1, _BN), lambda j: (0, j)),
        ],
        out_specs=[pl.BlockSpec((_B, _BN), lambda j: (0, j)),
                   pl.BlockSpec((_B, _BN), lambda j: (0, j))],
        out_shape=[jax.ShapeDtypeStruct((_B, _W1), jnp.float32),
                   jax.ShapeDtypeStruct((_B, _N), jnp.float32)],
    )(h, W2, b2, comp, ema)


def _tc_part2(h, W2, b2, comp, ema, exact_in):
    return pl.pallas_call(
        _mk_scores_body(_NB1, True),
        grid=(_NB2,),
        in_specs=[
            pl.BlockSpec((_B, _HH), lambda j: (0, 0)),
            pl.BlockSpec((_HH, _BN), lambda j: (0, j + _NB1)),
            pl.BlockSpec((1, _BN), lambda j: (0, j + _NB1)),
            pl.BlockSpec((1, _BN), lambda j: (0, j + _NB1)),
            pl.BlockSpec((1, _BN), lambda j: (0, j + _NB1)),
            pl.BlockSpec(memory_space=pl.ANY),
        ],
        out_specs=[pl.BlockSpec((_B, _BN), lambda j: (0, j)),
                   pl.BlockSpec((_B, _BN), lambda j: (0, j + _NB1))],
        out_shape=[jax.ShapeDtypeStruct((_B, _W2C), jnp.float32),
                   jax.ShapeDtypeStruct((_B, _N), jnp.float32)],
        input_output_aliases={5: 1},
    )(h, W2, b2, comp, ema, exact_in)


_GDN = lax.GatherDimensionNumbers(offset_dims=(), collapsed_slice_dims=(0,),
                                  start_index_map=(0,))


def _shuf(x, idx):
    return lax.gather(x, idx[:, None], dimension_numbers=_GDN,
                      slice_sizes=(1,),
                      mode=lax.GatherScatterMode.PROMISE_IN_BOUNDS)


def _pair_max(m, mi, x, xi):
    upd = x > m
    return jnp.where(upd, x, m), jnp.where(upd, xi, mi)


def _cross_lane_argmax(lane, v, i):
    for s in (8, 4, 2, 1):
        p = lane ^ s
        zv = _shuf(v, p)
        zi = _shuf(i, p)
        take = jnp.logical_or(zv > v, jnp.logical_and(zv == v, zi < i))
        v = jnp.where(take, zv, v)
        i = jnp.where(take, zi, i)
    return v[0], i[0]


def _lane_read(lane, vec, l, sentinel):
    m = jnp.where(lane == l, vec, sentinel)
    for s in (8, 4, 2, 1):
        m = jnp.maximum(m, _shuf(m, lane ^ s))
    return m[0]


def _row_id():
    return lax.axis_index("s") * 2 + lax.axis_index("c")


def _mk_sc_part_body(W, off):
    G = W // _CH
    J = G // 16

    def body(scores_hbm, val_hbm, idx_hbm, row_v, m1_v, m2v_v, m2i_v,
             val_v, idx_v):
        row = _row_id()
        neg = jnp.float32(-jnp.inf)
        lane = lax.broadcasted_iota(jnp.int32, (16,), 0)

        pltpu.sync_copy(scores_hbm.at[pl.ds(row * W, W)], row_v)

        def m1_body(g, c):
            base = g * _CH
            m = row_v[pl.ds(base, 16)]
            for k in range(1, 16):
                m = jnp.maximum(m, row_v[pl.ds(base + k * 16, 16)])
            m1_v[pl.ds(g * 16, 16)] = m
            return c

        lax.fori_loop(0, G, m1_body, 0)

        def m2_body(j, c):
            m = m1_v[pl.ds(j * 256, 16)]
            mi = jnp.full((16,), 0, jnp.int32) + j * 16
            for k in range(1, 16):
                m, mi = _pair_max(m, mi, m1_v[pl.ds(j * 256 + k * 16, 16)],
                                  jnp.full((16,), 0, jnp.int32) + (j * 16 + k))
            m2v_v[pl.ds(j * 16, 16)] = m
            m2i_v[pl.ds(j * 16, 16)] = mi
            return c

        lax.fori_loop(0, J, m2_body, 0)

        def extract(t, c):
            m = m2v_v[pl.ds(0, 16)]
            mi = m2i_v[pl.ds(0, 16)]
            for j in range(1, J):
                m, mi = _pair_max(m, mi, m2v_v[pl.ds(j * 16, 16)],
                                  m2i_v[pl.ds(j * 16, 16)])
            v, gbest = _cross_lane_argmax(lane, m, mi)
            vs = jnp.full((16,), v, jnp.float32)

            kb0 = gbest * _CH
            em = row_v[pl.ds(kb0, 16)]
            ei = kb0 + lane
            for k in range(1, 16):
                em, ei = _pair_max(em, ei, row_v[pl.ds(kb0 + k * 16, 16)],
                                   kb0 + k * 16 + lane)
            _, flat = _cross_lane_argmax(lane, em, ei)

            slot = (t >> 4) << 4
            iv = idx_v[pl.ds(slot, 16)]
            idx_v[pl.ds(slot, 16)] = jnp.where(lane == (t & 15),
                                               flat + off, iv)
            vv = val_v[pl.ds(slot, 16)]
            val_v[pl.ds(slot, 16)] = jnp.where(lane == (t & 15), v, vv)

            kb = (flat >> 4) << 4
            vreg = row_v[pl.ds(kb, 16)]
            row_v[pl.ds(kb, 16)] = jnp.where(lane == (flat & 15), neg, vreg)

            m = row_v[pl.ds(kb0, 16)]
            for k in range(1, 16):
                m = jnp.maximum(m, row_v[pl.ds(kb0 + k * 16, 16)])
            m1_v[pl.ds(gbest * 16, 16)] = m

            jb = gbest >> 4
            m = m1_v[pl.ds(jb * 256, 16)]
            mi = jnp.full((16,), 0, jnp.int32) + jb * 16
            for k in range(1, 16):
                m, mi = _pair_max(m, mi, m1_v[pl.ds(jb * 256 + k * 16, 16)],
                                  jb * 16 + k + jnp.full((16,), 0, jnp.int32))
            m2v_v[pl.ds(jb * 16, 16)] = m
            m2i_v[pl.ds(jb * 16, 16)] = mi
            return c

        lax.fori_loop(0, _K, extract, 0)

        pltpu.sync_copy(val_v, val_hbm.at[pl.ds(row * _K, _K)])
        pltpu.sync_copy(idx_v, idx_hbm.at[pl.ds(row * _K, _K)])

    return body, G, J


@functools.lru_cache(maxsize=4)
def _sc_part(W, off):
    body, G, J = _mk_sc_part_body(W, off)
    return pl.kernel(
        body,
        out_type=(jax.ShapeDtypeStruct((_B * _K,), jnp.float32),
                  jax.ShapeDtypeStruct((_B * _K,), jnp.int32)),
        mesh=plsc.VectorSubcoreMesh(core_axis_name="c", subcore_axis_name="s",
                                    num_cores=2, num_subcores=16),
        scratch_types=[
            pltpu.VMEM((W,), jnp.float32),
            pltpu.VMEM((G * 16,), jnp.float32),
            pltpu.VMEM((J * 16,), jnp.float32),
            pltpu.VMEM((J * 16,), jnp.int32),
            pltpu.VMEM((_K,), jnp.float32),
            pltpu.VMEM((_K,), jnp.int32),
        ],
    )


_MASKP = 100096


def _sc_merge_body(v1_hbm, i1_hbm, v2_hbm, i2_hbm, mask_hbm, sel_hbm,
                   candv_v, candi_v, mask_v, idx_v):
    row = _row_id()
    neg = jnp.float32(-jnp.inf)
    lane = lax.broadcasted_iota(jnp.int32, (16,), 0)

    pltpu.sync_copy(v1_hbm.at[pl.ds(row * _K, _K)], candv_v.at[pl.ds(0, _K)])
    pltpu.sync_copy(v2_hbm.at[pl.ds(row * _K, _K)], candv_v.at[pl.ds(_K, _K)])
    pltpu.sync_copy(i1_hbm.at[pl.ds(row * _K, _K)], candi_v.at[pl.ds(0, _K)])
    pltpu.sync_copy(i2_hbm.at[pl.ds(row * _K, _K)], candi_v.at[pl.ds(_K, _K)])

    def ext(t, c):
        m = candv_v[pl.ds(0, 16)]
        mp = lane
        for k in range(1, 8):
            m, mp = _pair_max(m, mp, candv_v[pl.ds(k * 16, 16)],
                              k * 16 + lane)
        _, pos = _cross_lane_argmax(lane, m, mp)

        pb = (pos >> 4) << 4
        giv = candi_v[pl.ds(pb, 16)]
        gidx = _lane_read(lane, giv, pos & 15, jnp.int32(-2147483648))

        slot = (t >> 4) << 4
        iv = idx_v[pl.ds(slot, 16)]
        idx_v[pl.ds(slot, 16)] = jnp.where(lane == (t & 15), gidx, iv)

        cv = candv_v[pl.ds(pb, 16)]
        candv_v[pl.ds(pb, 16)] = jnp.where(lane == (pos & 15), neg, cv)
        return c

    lax.fori_loop(0, _K, ext, 0)

    zero_v = jnp.zeros((16,), jnp.float32)

    def zero_body(i, c):
        base = i * _CH
        for k in range(16):
            mask_v[pl.ds(base + k * 16, 16)] = zero_v
        return c

    lax.fori_loop(0, _MASKP // _CH, zero_body, 0)

    def ones_body(t, c):
        iv = idx_v[pl.ds((t >> 4) << 4, 16)]
        flat = _lane_read(lane, iv, t & 15, jnp.int32(-2147483648))
        kb = (flat >> 4) << 4
        vreg = mask_v[pl.ds(kb, 16)]
        mask_v[pl.ds(kb, 16)] = jnp.where(lane == (flat & 15),
                                          jnp.float32(1.0), vreg)
        return c

    lax.fori_loop(0, _K, ones_body, 0)
    pltpu.sync_copy(mask_v.at[pl.ds(0, _N)], mask_hbm.at[pl.ds(row * _N, _N)])

    @pl.when(row == 0)
    def _():
        pltpu.sync_copy(idx_v, sel_hbm)


@functools.lru_cache(maxsize=1)
def _sc_merge():
    return pl.kernel(
        _sc_merge_body,
        out_type=(jax.ShapeDtypeStruct((_B * _N,), jnp.float32),
                  jax.ShapeDtypeStruct((_K,), jnp.int32)),
        mesh=plsc.VectorSubcoreMesh(core_axis_name="c", subcore_axis_name="s",
                                    num_cores=2, num_subcores=16),
        scratch_types=[
            pltpu.VMEM((2 * _K,), jnp.float32),
            pltpu.VMEM((2 * _K,), jnp.int32),
            pltpu.VMEM((_MASKP,), jnp.float32),
            pltpu.VMEM((_K,), jnp.int32),
        ],
    )


def kernel(z, W1, b1, ln_g, ln_b, W2, b2, competence, activation_ema):
    h = z @ W1 + b1
    mu = jnp.mean(h, axis=-1, keepdims=True)
    var = jnp.mean((h - mu) ** 2, axis=-1, keepdims=True)
    h = (h - mu) / jnp.sqrt(var + 1e-5) * ln_g + ln_b
    h = jax.nn.gelu(h, approximate=False)

    b2r = b2.reshape(1, _N)
    compr = competence.reshape(1, _N)
    emar = activation_ema.reshape(1, _N)
    p1, e1 = _tc_part1(h, W2, b2r, compr, emar)
    p2, e2 = _tc_part2(h, W2, b2r, compr, emar, e1)
    v1, i1 = _sc_part(_W1, 0)(p1.reshape(_B * _W1))
    v2, i2 = _sc_part(_W2C, _W1)(p2.reshape(_B * _W2C))
    mask_flat, sel = _sc_merge()(v1, i1, v2, i2)
    return mask_flat.reshape(_B, _N), sel, e2


# speedup vs baseline: 1.0777x; 1.0777x over previous
import functools

import jax
import jax.numpy as jnp
import numpy as np
from jax import lax
from jax.experimental import pallas as pl
from jax.experimental.pallas import tpu as pltpu
from jax.experimental.pallas import tpu_sc as plsc

_B = 32
_H = 1024
_HH = 512
_N = 100000
_K = 64
_BN = 4096
_NBLK = (_N + _BN - 1) // _BN

_NP = _NBLK * _BN
_CH = 256
_G = _NP // _CH
_GP = _G
_J = _GP // 16


def _scores_body(h_ref, w2_ref, b2_ref, comp_ref, ema_ref,
                 out_ref, exact_ref):
    s = jnp.dot(h_ref[...], w2_ref[...],
                preferred_element_type=jnp.float32) + b2_ref[...]
    s = s + comp_ref[...] * 0.3 + (1.0 / (ema_ref[...] + 1e-6)) * 0.1
    col = pl.program_id(0) * _BN + lax.broadcasted_iota(jnp.int32, (_B, _BN), 1)
    out_ref[...] = jnp.where(col < _N, s, -jnp.inf)
    exact_ref[...] = s


def _tc_scores(h, W2, b2, comp, ema):
    return pl.pallas_call(
        _scores_body,
        grid=(_NBLK,),
        in_specs=[
            pl.BlockSpec((_B, _HH), lambda j: (0, 0)),
            pl.BlockSpec((_HH, _BN), lambda j: (0, j)),
            pl.BlockSpec((1, _BN), lambda j: (0, j)),
            pl.BlockSpec((1, _BN), lambda j: (0, j)),
            pl.BlockSpec((1, _BN), lambda j: (0, j)),
        ],
        out_specs=[pl.BlockSpec((_B, _BN), lambda j: (0, j)),
                   pl.BlockSpec((_B, _BN), lambda j: (0, j))],
        out_shape=[jax.ShapeDtypeStruct((_B, _NP), jnp.float32),
                   jax.ShapeDtypeStruct((_B, _N), jnp.float32)],
    )(h, W2, b2, comp, ema)


_GDN = lax.GatherDimensionNumbers(offset_dims=(), collapsed_slice_dims=(0,),
                                  start_index_map=(0,))


def _shuf(x, idx):
    return lax.gather(x, idx[:, None], dimension_numbers=_GDN,
                      slice_sizes=(1,),
                      mode=lax.GatherScatterMode.PROMISE_IN_BOUNDS)


def _pair_max(m, mi, x, xi):
    upd = x > m
    return jnp.where(upd, x, m), jnp.where(upd, xi, mi)


def _cross_lane_argmax(lane, v, i):
    for s in (8, 4, 2, 1):
        p = lane ^ s
        zv = _shuf(v, p)
        zi = _shuf(i, p)
        take = jnp.logical_or(zv > v, jnp.logical_and(zv == v, zi < i))
        v = jnp.where(take, zv, v)
        i = jnp.where(take, zi, i)
    return v[0], i[0]


def _sc_body(scores_hbm, mask_hbm, sel_hbm, row_v, m1_v, m2v_v, m2i_v, idx_v):
    row = lax.axis_index("s") * 2 + lax.axis_index("c")
    neg = jnp.float32(-jnp.inf)
    neg_v = jnp.full((16,), neg, jnp.float32)
    lane = lax.broadcasted_iota(jnp.int32, (16,), 0)

    pltpu.sync_copy(scores_hbm.at[pl.ds(row * _NP, _NP)], row_v)

    def m1_body(g, c):
        base = g * _CH
        m = row_v[pl.ds(base, 16)]
        for k in range(1, 16):
            m = jnp.maximum(m, row_v[pl.ds(base + k * 16, 16)])
        m1_v[pl.ds(g * 16, 16)] = m
        return c

    lax.fori_loop(0, _G, m1_body, 0)
    for g in range(_G, _GP):
        m1_v[pl.ds(g * 16, 16)] = neg_v

    def m2_body(j, c):
        m = m1_v[pl.ds(j * 256, 16)]
        mi = jnp.full((16,), 0, jnp.int32) + j * 16
        for k in range(1, 16):
            m, mi = _pair_max(m, mi, m1_v[pl.ds(j * 256 + k * 16, 16)],
                              jnp.full((16,), 0, jnp.int32) + (j * 16 + k))
        m2v_v[pl.ds(j * 16, 16)] = m
        m2i_v[pl.ds(j * 16, 16)] = mi
        return c

    lax.fori_loop(0, _J, m2_body, 0)

    def extract(t, c):
        m = m2v_v[pl.ds(0, 16)]
        mi = m2i_v[pl.ds(0, 16)]
        for j in range(1, _J):
            m, mi = _pair_max(m, mi, m2v_v[pl.ds(j * 16, 16)],
                              m2i_v[pl.ds(j * 16, 16)])
        v, gbest = _cross_lane_argmax(lane, m, mi)
        vs = jnp.full((16,), v, jnp.float32)

        kb0 = gbest * _CH
        em = row_v[pl.ds(kb0, 16)]
        ei = kb0 + lane
        for k in range(1, 16):
            em, ei = _pair_max(em, ei, row_v[pl.ds(kb0 + k * 16, 16)],
                               kb0 + k * 16 + lane)
        _, flat = _cross_lane_argmax(lane, em, ei)

        slot = (t >> 4) << 4
        iv = idx_v[pl.ds(slot, 16)]
        idx_v[pl.ds(slot, 16)] = jnp.where(lane == (t & 15), flat, iv)

        kb = (flat >> 4) << 4
        vreg = row_v[pl.ds(kb, 16)]
        row_v[pl.ds(kb, 16)] = jnp.where(lane == (flat & 15), neg, vreg)

        m = row_v[pl.ds(kb0, 16)]
        for k in range(1, 16):
            m = jnp.maximum(m, row_v[pl.ds(kb0 + k * 16, 16)])
        m1_v[pl.ds(gbest * 16, 16)] = m

        jb = gbest >> 4
        m = m1_v[pl.ds(jb * 256, 16)]
        mi = jnp.full((16,), 0, jnp.int32) + jb * 16
        for k in range(1, 16):
            m, mi = _pair_max(m, mi, m1_v[pl.ds(jb * 256 + k * 16, 16)],
                              jb * 16 + k + jnp.full((16,), 0, jnp.int32))
        m2v_v[pl.ds(jb * 16, 16)] = m
        m2i_v[pl.ds(jb * 16, 16)] = mi
        return c

    lax.fori_loop(0, _K, extract, 0)

    zero_v = jnp.zeros((16,), jnp.float32)

    def zero_body(i, c):
        base = i * _CH
        for k in range(16):
            row_v[pl.ds(base + k * 16, 16)] = zero_v
        return c

    lax.fori_loop(0, _NP // _CH, zero_body, 0)

    def ones_body(t, c):
        iv = idx_v[pl.ds((t >> 4) << 4, 16)]
        m = jnp.where(lane == (t & 15), iv, jnp.int32(-2147483648))
        for s in (8, 4, 2, 1):
            m = jnp.maximum(m, _shuf(m, lane ^ s))
        flat = m[0]
        kb = (flat >> 4) << 4
        vreg = row_v[pl.ds(kb, 16)]
        row_v[pl.ds(kb, 16)] = jnp.where(lane == (flat & 15),
                                         jnp.float32(1.0), vreg)
        return c

    lax.fori_loop(0, _K, ones_body, 0)
    pltpu.sync_copy(row_v, mask_hbm.at[pl.ds(row * _NP, _NP)])

    @pl.when(row == 0)
    def _():
        pltpu.sync_copy(idx_v, sel_hbm)


@functools.lru_cache(maxsize=1)
def _sc_topk():
    return pl.kernel(
        _sc_body,
        out_type=(jax.ShapeDtypeStruct((_B * _NP,), jnp.float32),
                  jax.ShapeDtypeStruct((_K,), jnp.int32)),
        mesh=plsc.VectorSubcoreMesh(core_axis_name="c", subcore_axis_name="s",
                                    num_cores=2, num_subcores=16),
        scratch_types=[
            pltpu.VMEM((_NP,), jnp.float32),
            pltpu.VMEM((_GP * 16,), jnp.float32),
            pltpu.VMEM((_J * 16,), jnp.float32),
            pltpu.VMEM((_J * 16,), jnp.int32),
            pltpu.VMEM((_K,), jnp.int32),
        ],
    )


def kernel(z, W1, b1, ln_g, ln_b, W2, b2, competence, activation_ema):
    h = z @ W1 + b1
    mu = jnp.mean(h, axis=-1, keepdims=True)
    var = jnp.mean((h - mu) ** 2, axis=-1, keepdims=True)
    h = (h - mu) / jnp.sqrt(var + 1e-5) * ln_g + ln_b
    h = jax.nn.gelu(h, approximate=False)
    padded, exact = _tc_scores(h, W2, b2.reshape(1, _N),
                               competence.reshape(1, _N),
                               activation_ema.reshape(1, _N))
    mask_flat, sel = _sc_topk()(padded.reshape(_B * _NP))
    mask = mask_flat.reshape(_B, _NP)[:, :_N]
    return mask, sel, exact
